# baseline (speedup 1.0000x reference)
import functools
import jax
import jax.numpy as jnp
from jax import lax
from jax.experimental import pallas as pl
from jax.experimental.pallas import tpu as pltpu
from jax.experimental.pallas import tpu_sc as plsc

VOCAB = 100001
POS = 200
BATCH = 16384

NC = 2
NS = 16
NW = NC * NS

TOTAL = BATCH * POS
CHUNK = 4096
IDX_SLICES = CHUNK // 128
NCHUNKS_TOTAL = TOTAL // CHUNK
NCHUNK = NCHUNKS_TOTAL // NW


def _sc_kernel(x0_hbm, tab_hbm, out_hbm, xbuf, idxbuf, gbuf, sem):
    wid = lax.axis_index("s") * NC + lax.axis_index("c")
    chunk0 = wid * NCHUNK
    lane = lax.iota(jnp.int32, 16)

    def chunk_body(c, carry):
        base = (chunk0 + c) * CHUNK
        pltpu.sync_copy(x0_hbm.at[pl.ds(base, CHUNK)], xbuf)

        def idx_body(r, carry2):
            e0 = base + r * 128
            for u in range(8):
                e = e0 + u * 16 + lane
                off = lax.rem(e, POS) * VOCAB
                x = xbuf[pl.ds(r * 128 + u * 16, 16)]
                idxbuf[pl.ds(r * 128 + u * 16, 16)] = x + off
            return carry2

        lax.fori_loop(0, IDX_SLICES, idx_body, 0, unroll=False)

        def fire(i, carry2):
            pltpu.async_copy(
                tab_hbm.at[idxbuf.at[pl.ds(i * 128, 128)]],
                gbuf.at[pl.ds(i * 128, 128)],
                sem,
            )
            return carry2

        lax.fori_loop(0, IDX_SLICES, fire, 0, unroll=False)
        pltpu.make_async_copy(x0_hbm.at[pl.ds(base, CHUNK)], gbuf, sem).wait()

        pos0 = lax.rem(POS - 1 - lax.rem(base, POS) + POS, POS)
        for k in range(CHUNK // POS + 1):
            pos = pos0 + POS * k
            posc = jnp.minimum(pos, CHUNK - 1)
            a = lax.bitwise_and(posc, -16)
            m = (a + lane) == pos
            xv = xbuf[pl.ds(a, 16)]
            gv = gbuf[pl.ds(a, 16)]
            gbuf[pl.ds(a, 16)] = jnp.where(m, xv, gv)

        pltpu.sync_copy(gbuf, out_hbm.at[pl.ds(base, CHUNK)])
        return carry

    lax.fori_loop(0, NCHUNK, chunk_body, 0, unroll=False)


@jax.jit
def _scramble(x0v, tab):
    mesh = plsc.VectorSubcoreMesh(core_axis_name="c", subcore_axis_name="s")
    f = functools.partial(
        pl.kernel,
        out_type=jax.ShapeDtypeStruct(x0v.shape, jnp.int32),
        mesh=mesh,
        scratch_types=[
            pltpu.VMEM((CHUNK,), jnp.int32),
            pltpu.VMEM((CHUNK,), jnp.int32),
            pltpu.VMEM((CHUNK,), jnp.int32),
            pltpu.SemaphoreType.DMA,
        ],
    )(_sc_kernel)
    return f(x0v, tab)


def kernel(x0, x1, x2, scrambler):
    b, p = x0.shape
    x0v = x0.reshape(-1)
    tab = scrambler.reshape(-1)
    outv = _scramble(x0v, tab)
    return (outv.reshape(b, p), x1, x2)


# speedup vs baseline: 1.0474x; 1.0474x over previous
import functools
import jax
import jax.numpy as jnp
from jax import lax
from jax.experimental import pallas as pl
from jax.experimental.pallas import tpu as pltpu
from jax.experimental.pallas import tpu_sc as plsc

VOCAB = 100001
POS = 200
BATCH = 16384

NC = 2
NS = 16
NW = NC * NS

TOTAL = BATCH * POS
CHUNK = 12800
IDX_SLICES = CHUNK // 128
NCHUNK = TOTAL // CHUNK // NW
PERIOD = 400
NPATCH = CHUNK // POS
LAST_OFF = (POS - 1) * VOCAB


def _sc_kernel(x0_hbm, tab_hbm, out_hbm,
               xb0, xb1, ib0, ib1, gb0, gb1, offp,
               sem_in0, sem_in1, sem_g0, sem_g1, sem_out0, sem_out1):
    wid = lax.axis_index("s") * NC + lax.axis_index("c")
    elem0 = wid * (NCHUNK * CHUNK)
    lane = lax.iota(jnp.int32, 16)

    for j in range(PERIOD // 16):
        offp[pl.ds(j * 16, 16)] = lax.rem(j * 16 + lane, POS) * VOCAB

    def in_copy(c, xb, sem):
        pltpu.async_copy(
            x0_hbm.at[pl.ds(elem0 + c * CHUNK, CHUNK)], xb, sem)

    def wait_bytes(buf, sem):
        pltpu.make_async_copy(x0_hbm.at[pl.ds(0, CHUNK)], buf, sem).wait()

    def compute_idx(xb, ib):
        def body(q, carry):
            s0 = q * PERIOD
            for u in range(PERIOD // 16):
                s = s0 + u * 16
                ib[pl.ds(s, 16)] = xb[pl.ds(s, 16)] + offp[pl.ds(u * 16, 16)]
            return carry

        lax.fori_loop(0, CHUNK // PERIOD, body, 0, unroll=False)

    def fire(ib, gb, sem):
        def body(i, carry):
            pltpu.async_copy(
                tab_hbm.at[ib.at[pl.ds(i * 128, 128)]],
                gb.at[pl.ds(i * 128, 128)],
                sem,
            )
            return carry

        lax.fori_loop(0, IDX_SLICES, body, 0, unroll=False)

    def patch(ib, gb):
        for k in range(NPATCH):
            pos = POS - 1 + POS * k
            a = pos & ~15
            m = lane == (pos - a)
            iv = ib[pl.ds(a, 16)] - LAST_OFF
            gv = gb[pl.ds(a, 16)]
            gb[pl.ds(a, 16)] = jnp.where(m, iv, gv)

    def out_copy(c, gb, sem):
        pltpu.async_copy(
            gb, out_hbm.at[pl.ds(elem0 + c * CHUNK, CHUNK)], sem)

    xbs = (xb0, xb1)
    ibs = (ib0, ib1)
    gbs = (gb0, gb1)
    sin = (sem_in0, sem_in1)
    sg = (sem_g0, sem_g1)
    sout = (sem_out0, sem_out1)

    in_copy(0, xb0, sem_in0)
    in_copy(1, xb1, sem_in1)
    wait_bytes(xb0, sem_in0)
    compute_idx(xb0, ib0)
    in_copy(2, xb0, sem_in0)

    for c in range(NCHUNK):
        b = c & 1
        if c >= 2:
            wait_bytes(gbs[b], sout[b])
        fire(ibs[b], gbs[b], sg[b])
        if c + 1 < NCHUNK:
            nb = (c + 1) & 1
            wait_bytes(xbs[nb], sin[nb])
            compute_idx(xbs[nb], ibs[nb])
            if c + 3 < NCHUNK:
                in_copy(c + 3, xbs[nb], sin[nb])
        wait_bytes(gbs[b], sg[b])
        patch(ibs[b], gbs[b])
        out_copy(c, gbs[b], sout[b])

    wait_bytes(gbs[(NCHUNK - 2) & 1], sout[(NCHUNK - 2) & 1])
    wait_bytes(gbs[(NCHUNK - 1) & 1], sout[(NCHUNK - 1) & 1])


@jax.jit
def _scramble(x0v, tab):
    mesh = plsc.VectorSubcoreMesh(core_axis_name="c", subcore_axis_name="s")
    f = functools.partial(
        pl.kernel,
        out_type=jax.ShapeDtypeStruct(x0v.shape, jnp.int32),
        mesh=mesh,
        scratch_types=[
            pltpu.VMEM((CHUNK,), jnp.int32),
            pltpu.VMEM((CHUNK,), jnp.int32),
            pltpu.VMEM((CHUNK,), jnp.int32),
            pltpu.VMEM((CHUNK,), jnp.int32),
            pltpu.VMEM((CHUNK,), jnp.int32),
            pltpu.VMEM((CHUNK,), jnp.int32),
            pltpu.VMEM((PERIOD,), jnp.int32),
            pltpu.SemaphoreType.DMA,
            pltpu.SemaphoreType.DMA,
            pltpu.SemaphoreType.DMA,
            pltpu.SemaphoreType.DMA,
            pltpu.SemaphoreType.DMA,
            pltpu.SemaphoreType.DMA,
        ],
    )(_sc_kernel)
    return f(x0v, tab)


def kernel(x0, x1, x2, scrambler):
    b, p = x0.shape
    x0v = x0.reshape(-1)
    tab = scrambler.reshape(-1)
    outv = _scramble(x0v, tab)
    return (outv.reshape(b, p), x1, x2)
